# baseline (speedup 1.0000x reference)
import functools

import jax
import jax.numpy as jnp
from jax import lax
from jax.experimental import pallas as pl
from jax.experimental.pallas import tpu as pltpu
from jax.experimental.pallas import tpu_sc as plsc

T = 2048
D = 1024
V = 32000
E = 16
F = 1024
H = 16
DH = 64
C = 160
S = E * C
PAD = 8
NEG = -1e9

NW = 32


def _sc_gather(table, idx, n_rows, d):
    b_per_w = n_rows // NW
    mesh = plsc.VectorSubcoreMesh(core_axis_name="c", subcore_axis_name="s")

    @functools.partial(
        pl.kernel,
        out_type=jax.ShapeDtypeStruct((n_rows, d), jnp.float32),
        mesh=mesh,
        scratch_types=[
            pltpu.VMEM((b_per_w,), jnp.int32),
            pltpu.VMEM((b_per_w, d), jnp.float32),
            pltpu.SemaphoreType.DMA,
        ],
    )
    def k(table_hbm, idx_hbm, out_hbm, idx_v, rows_v, sem):
        wid = lax.axis_index("s") * 2 + lax.axis_index("c")
        base = wid * b_per_w
        pltpu.sync_copy(idx_hbm.at[pl.ds(base, b_per_w)], idx_v)
        pltpu.async_copy(table_hbm.at[idx_v], rows_v, sem).wait()
        pltpu.sync_copy(rows_v, out_hbm.at[pl.ds(base, b_per_w)])

    return k(table, idx)


def _norm0_body(x_ref, ln_ref, hm_ref):
    x = x_ref[...]
    rs = lax.rsqrt(jnp.mean(x * x, axis=-1, keepdims=True) + 1e-6)
    hm_ref[...] = x * rs * ln_ref[...]


def _norm1_body(x_ref, y_ref, s_ref, ln_ref, xo_ref, hm_ref):
    x = x_ref[...] + y_ref[...] * s_ref[...]
    xo_ref[...] = x
    rs = lax.rsqrt(jnp.mean(x * x, axis=-1, keepdims=True) + 1e-6)
    hm_ref[...] = x * rs * ln_ref[...]


def _qkv_body(hm_ref, wq_ref, wk_ref, wv_ref, q_ref, k_ref, v_ref):
    hm = hm_ref[...]
    for w_ref, o_ref in ((wq_ref, q_ref), (wk_ref, k_ref), (wv_ref, v_ref)):
        o_ref[0] = jnp.dot(hm, w_ref[0], preferred_element_type=jnp.float32)


_QB = 1024


def _attn_body(q_ref, k_ref, v_ref, o_ref):
    qi = pl.program_id(1)
    s = lax.dot_general(q_ref[0], k_ref[0], (((1,), (1,)), ((), ())),
                        preferred_element_type=jnp.float32)
    s = s * (1.0 / (DH ** 0.5))
    row = qi * _QB + lax.broadcasted_iota(jnp.int32, (_QB, T), 0)
    col = lax.broadcasted_iota(jnp.int32, (_QB, T), 1)
    s = s + jnp.where(col <= row, 0.0, NEG)
    m = jnp.max(s, axis=-1, keepdims=True)
    u = jnp.exp(s - m)
    uv = jnp.dot(u, v_ref[0], preferred_element_type=jnp.float32)
    r = jnp.sum(u, axis=-1, keepdims=True)
    o_ref[0] = uv * pl.reciprocal(r, approx=True)


def _post_body(o_ref, wo_ref, x_ref, ln2_ref, wr_ref,
               xo_ref, hm_ref, probs_ref):
    h = pl.program_id(0)
    part = jnp.dot(o_ref[0], wo_ref[0], preferred_element_type=jnp.float32)

    @pl.when(h == 0)
    def _():
        xo_ref[...] = x_ref[...] + part

    @pl.when(h > 0)
    def _():
        xo_ref[...] += part

    @pl.when(h == H - 1)
    def _():
        x = xo_ref[...]
        rs = lax.rsqrt(jnp.mean(x * x, axis=-1, keepdims=True) + 1e-6)
        hm = x * rs * ln2_ref[...]
        hm_ref[:T, :] = hm
        hm_ref[T:, :] = jnp.zeros((PAD, D), jnp.float32)
        logits = jnp.dot(hm, wr_ref[...], preferred_element_type=jnp.float32)
        m = jnp.max(logits, axis=-1, keepdims=True)
        p = jnp.exp(logits - m)
        probs_ref[...] = p / jnp.sum(p, axis=-1, keepdims=True)


_TB = 256
_NTB = T // _TB


def _route_body(probs_ref, slot_idx_ref, comb_ref, scale_ref, aux_ref,
                cnt_ref, fsum_ref, psum_ref):
    i = pl.program_id(0)

    @pl.when(i == 0)
    def _():
        cnt_ref[...] = jnp.zeros((1, E), jnp.float32)
        fsum_ref[...] = jnp.zeros((1, E), jnp.float32)
        psum_ref[...] = jnp.zeros((1, E), jnp.float32)
        slot_idx_ref[...] = jnp.full((1, S), T, jnp.int32)

    probs = probs_ref[...]
    gate = jnp.max(probs, axis=-1, keepdims=True)
    e_iota = lax.broadcasted_iota(jnp.int32, (_TB, E), 1)
    expert = jnp.min(jnp.where(probs == gate, e_iota, E), axis=-1,
                     keepdims=True)
    oh = (e_iota == expert).astype(jnp.float32)
    ti = lax.broadcasted_iota(jnp.int32, (_TB, _TB), 0)
    tj = lax.broadcasted_iota(jnp.int32, (_TB, _TB), 1)
    tril = (tj <= ti).astype(jnp.float32)
    cnt = jnp.dot(tril, oh, preferred_element_type=jnp.float32)
    cnt = cnt + cnt_ref[...]
    pos = (jnp.sum(cnt * oh, axis=-1, keepdims=True) - 1.0).astype(jnp.int32)
    cnt_ref[...] += jnp.sum(oh, axis=0, keepdims=True)
    fsum_ref[...] += jnp.sum(oh, axis=0, keepdims=True)
    psum_ref[...] += jnp.sum(probs, axis=0, keepdims=True)

    keep = pos < C
    posc = jnp.minimum(pos, C - 1)
    slot = expert * C + posc
    comb_ref[...] = slot
    scale_ref[...] = jnp.where(keep, gate, 0.0)
    slot_k = jnp.where(keep, slot, S + 1)
    s_iota = lax.broadcasted_iota(jnp.int32, (_TB, S), 1)
    t_iota = i * _TB + lax.broadcasted_iota(jnp.int32, (_TB, S), 0)
    hit = s_iota == slot_k
    blk_min = jnp.min(jnp.where(hit, t_iota, T), axis=0, keepdims=True)
    slot_idx_ref[...] = jnp.minimum(slot_idx_ref[...], blk_min)

    @pl.when(i == _NTB - 1)
    def _():
        f = fsum_ref[...] * (1.0 / T)
        pbar = psum_ref[...] * (1.0 / T)
        aux_ref[...] = jnp.sum(f * pbar, axis=-1, keepdims=True) * E


def _ffn_body(xd_ref, w1_ref, w2_ref, yd_ref):
    h = jnp.dot(xd_ref[0], w1_ref[0], preferred_element_type=jnp.float32)
    h = jnp.maximum(h, 0.0)
    yd_ref[0] = jnp.dot(h, w2_ref[0], preferred_element_type=jnp.float32)


_VT = 1280


def _lmhead_body(x_ref, y_ref, s_ref, emb_ref, out_ref):
    xf = (x_ref[...] + y_ref[...] * s_ref[...]).astype(jnp.bfloat16)
    eb = emb_ref[...].astype(jnp.bfloat16)
    out_ref[...] = lax.dot_general(xf, eb, (((1,), (1,)), ((), ())),
                                   preferred_element_type=jnp.float32)


def _f32(shape):
    return jax.ShapeDtypeStruct(shape, jnp.float32)


def _bf16(shape):
    return jax.ShapeDtypeStruct(shape, jnp.bfloat16)


def kernel(input_ids, embed_tokens, ln1, Wq, Wk, Wv, Wo, ln2, Wr, W1, W2):
    ids = input_ids.reshape(T).astype(jnp.int32)
    x = _sc_gather(embed_tokens, ids, T, D)

    WqT = Wq.reshape(2, D, H, DH).transpose(0, 2, 1, 3)
    WkT = Wk.reshape(2, D, H, DH).transpose(0, 2, 1, 3)
    WvT = Wv.reshape(2, D, H, DH).transpose(0, 2, 1, 3)
    WoR = Wo.reshape(2, H, DH, D)

    aux_total = jnp.float32(0.0)
    y = None
    scale = None
    for l in range(2):
        ln1_l = ln1[l].reshape(1, D)
        ln2_l = ln2[l].reshape(1, D)
        if l == 0:
            hm1 = pl.pallas_call(
                _norm0_body, out_shape=_f32((T, D)),
            )(x, ln1_l)
        else:
            x, hm1 = pl.pallas_call(
                _norm1_body, out_shape=(_f32((T, D)), _f32((T, D))),
            )(x, y, scale, ln1_l)

        q, k, v = pl.pallas_call(
            _qkv_body,
            grid=(H,),
            in_specs=[
                pl.BlockSpec((T, D), lambda h: (0, 0)),
                pl.BlockSpec((1, D, DH), lambda h: (h, 0, 0)),
                pl.BlockSpec((1, D, DH), lambda h: (h, 0, 0)),
                pl.BlockSpec((1, D, DH), lambda h: (h, 0, 0)),
            ],
            out_specs=[
                pl.BlockSpec((1, T, DH), lambda h: (h, 0, 0)),
                pl.BlockSpec((1, T, DH), lambda h: (h, 0, 0)),
                pl.BlockSpec((1, T, DH), lambda h: (h, 0, 0)),
            ],
            out_shape=(_f32((H, T, DH)), _f32((H, T, DH)),
                       _f32((H, T, DH))),
        )(hm1, WqT[l], WkT[l], WvT[l])

        o = pl.pallas_call(
            _attn_body,
            grid=(H, T // _QB),
            in_specs=[
                pl.BlockSpec((1, _QB, DH), lambda h, qi: (h, qi, 0)),
                pl.BlockSpec((1, T, DH), lambda h, qi: (h, 0, 0)),
                pl.BlockSpec((1, T, DH), lambda h, qi: (h, 0, 0)),
            ],
            out_specs=pl.BlockSpec((1, _QB, DH), lambda h, qi: (h, qi, 0)),
            out_shape=_f32((H, T, DH)),
        )(q, k, v)

        x, hm_pad, probs = pl.pallas_call(
            _post_body,
            grid=(H,),
            in_specs=[
                pl.BlockSpec((1, T, DH), lambda h: (h, 0, 0)),
                pl.BlockSpec((1, DH, D), lambda h: (h, 0, 0)),
                pl.BlockSpec((T, D), lambda h: (0, 0)),
                pl.BlockSpec((1, D), lambda h: (0, 0)),
                pl.BlockSpec((D, E), lambda h: (0, 0)),
            ],
            out_specs=[
                pl.BlockSpec((T, D), lambda h: (0, 0)),
                pl.BlockSpec((T + PAD, D), lambda h: (0, 0)),
                pl.BlockSpec((T, E), lambda h: (0, 0)),
            ],
            out_shape=(_f32((T, D)), _f32((T + PAD, D)), _f32((T, E))),
        )(o, WoR[l], x, ln2_l, Wr[l])

        slot_idx, comb_idx, scale, aux = pl.pallas_call(
            _route_body,
            grid=(_NTB,),
            in_specs=[pl.BlockSpec((_TB, E), lambda i: (i, 0))],
            out_specs=[
                pl.BlockSpec((1, S), lambda i: (0, 0)),
                pl.BlockSpec((_TB, 1), lambda i: (i, 0)),
                pl.BlockSpec((_TB, 1), lambda i: (i, 0)),
                pl.BlockSpec((1, 1), lambda i: (0, 0)),
            ],
            out_shape=(
                jax.ShapeDtypeStruct((1, S), jnp.int32),
                jax.ShapeDtypeStruct((T, 1), jnp.int32),
                _f32((T, 1)),
                _f32((1, 1)),
            ),
            scratch_shapes=[
                pltpu.VMEM((1, E), jnp.float32),
                pltpu.VMEM((1, E), jnp.float32),
                pltpu.VMEM((1, E), jnp.float32),
            ],
        )(probs)
        aux_total = aux_total + aux[0, 0]

        xd = _sc_gather(hm_pad, slot_idx.reshape(S), S, D)
        yd = pl.pallas_call(
            _ffn_body,
            grid=(E,),
            in_specs=[
                pl.BlockSpec((1, C, D), lambda e: (e, 0, 0)),
                pl.BlockSpec((1, D, F), lambda e: (e, 0, 0)),
                pl.BlockSpec((1, F, D), lambda e: (e, 0, 0)),
            ],
            out_specs=pl.BlockSpec((1, C, D), lambda e: (e, 0, 0)),
            out_shape=_f32((E, C, D)),
        )(xd.reshape(E, C, D), W1[l], W2[l])

        y = _sc_gather(yd.reshape(S, D), comb_idx.reshape(T), T, D)

    logits = pl.pallas_call(
        _lmhead_body,
        grid=(V // _VT,),
        in_specs=[
            pl.BlockSpec((T, D), lambda i: (0, 0)),
            pl.BlockSpec((T, D), lambda i: (0, 0)),
            pl.BlockSpec((T, 1), lambda i: (0, 0)),
            pl.BlockSpec((_VT, D), lambda i: (i, 0)),
        ],
        out_specs=pl.BlockSpec((T, _VT), lambda i: (0, i)),
        out_shape=_f32((T, V)),
    )(x, y, scale, embed_tokens)

    return logits.reshape(1, T, V), 0.01 * aux_total
